# Initial kernel scaffold; baseline (speedup 1.0000x reference)
import jax
import jax.numpy as jnp
from jax.experimental import pallas as pl


def kernel(queries, keys):
    raise NotImplementedError("write your pallas kernel here")


# speedup vs baseline: 3.7855x; 3.7855x over previous
import jax
import jax.numpy as jnp
from jax.experimental import pallas as pl
from jax.experimental.pallas import tpu as pltpu

_QB = 392
_KB = 1024
_D = 128
_LANES = 128


def _knn_kernel(q_ref, k_ref, ps_ref, im_ref, acc_ref, ksq_ref, *, nk, nq):
    j = pl.program_id(0)
    i = pl.program_id(1)

    k = k_ref[...]

    @pl.when(i == 0)
    def _ksq():
        kf = k.astype(jnp.float32)
        kk = (kf * kf).astype(jnp.bfloat16)
        ones = jnp.ones((8, _D), jnp.bfloat16)
        ksq_ref[...] = jax.lax.dot_general(
            ones, kk, (((1,), (1,)), ((), ())),
            preferred_element_type=jnp.float32)

    q = q_ref[...]
    dot = jax.lax.dot_general(
        q, k, (((1,), (1,)), ((), ())),
        preferred_element_type=jnp.float32)
    cand = dot + ksq_ref[0:1, :]
    m = cand[:, 0:_LANES]
    for c in range(1, _KB // _LANES):
        m = jnp.minimum(m, cand[:, c * _LANES:(c + 1) * _LANES])

    sl = pl.ds(i * _QB, _QB)

    @pl.when(j == 0)
    def _first():
        acc_ref[sl, :] = m

    @pl.when(j > 0)
    def _rest():
        acc_ref[sl, :] = jnp.minimum(acc_ref[sl, :], m)

    @pl.when(j == nk - 1)
    def _epilogue():
        qf = q.astype(jnp.float32)
        q_sq = 0.25 * jnp.sum(qf * qf, axis=1)
        mn = jnp.min(acc_ref[sl, :], axis=1)
        d2 = jnp.maximum(q_sq + mn, 0.0)
        ps = jnp.sqrt(d2 + 1e-12)
        ps_ref[...] = ps.reshape(1, 1, _QB)
        tmax = jnp.broadcast_to(jnp.max(ps), (1, 1, _LANES))

        @pl.when(i % 2 == 0)
        def _im_first():
            im_ref[...] = tmax

        @pl.when(i % 2 == 1)
        def _im_rest():
            im_ref[...] = jnp.maximum(im_ref[...], tmax)


def kernel(queries, keys):
    Q, D = queries.shape
    K, _ = keys.shape
    n_img = 4
    nq = Q // _QB
    nk = (K + _KB - 1) // _KB
    kp = nk * _KB
    keys_p = jnp.pad(keys, ((0, kp - K), (0, 0)), constant_values=1000.0)
    kb = keys_p.astype(jnp.bfloat16)
    qb = (-2.0 * queries).astype(jnp.bfloat16)

    ps3, im3 = pl.pallas_call(
        lambda qr, kr, pr, ir, ar, sr: _knn_kernel(
            qr, kr, pr, ir, ar, sr, nk=nk, nq=nq),
        grid=(nk, nq),
        in_specs=[
            pl.BlockSpec((_QB, D), lambda j, i: (i, 0)),
            pl.BlockSpec((_KB, D), lambda j, i: (j, 0)),
        ],
        out_specs=[
            pl.BlockSpec((1, 1, _QB), lambda j, i: (i, 0, 0)),
            pl.BlockSpec((1, 1, _LANES), lambda j, i: (i // 2, 0, 0)),
        ],
        out_shape=[
            jax.ShapeDtypeStruct((nq, 1, _QB), jnp.float32),
            jax.ShapeDtypeStruct((n_img, 1, _LANES), jnp.float32),
        ],
        scratch_shapes=[
            pltpu.VMEM((Q, _LANES), jnp.float32),
            pltpu.VMEM((8, _KB), jnp.float32),
        ],
    )(qb, kb)

    patch_scores = ps3.reshape(-1)
    image_scores = im3[:, 0, 0].reshape(n_img)
    return image_scores, patch_scores


# speedup vs baseline: 4.0280x; 1.0641x over previous
import jax
import jax.numpy as jnp
from jax.experimental import pallas as pl
from jax.experimental.pallas import tpu as pltpu

_QB = 784
_KB = 1024
_DA = 130
_LANES = 128


def _knn_kernel(q_ref, k_ref, ps_ref, im_ref, acc_ref, *, nk):
    j = pl.program_id(0)
    i = pl.program_id(1)

    q = q_ref[...]
    k = k_ref[...]
    cand = jax.lax.dot_general(
        q, k, (((1,), (1,)), ((), ())),
        preferred_element_type=jnp.float32)
    m = cand[:, 0:_LANES]
    for c in range(1, _KB // _LANES):
        m = jnp.minimum(m, cand[:, c * _LANES:(c + 1) * _LANES])

    sl = pl.ds(i * _QB, _QB)

    @pl.when(j == 0)
    def _first():
        acc_ref[sl, :] = m

    @pl.when(j > 0)
    def _rest():
        acc_ref[sl, :] = jnp.minimum(acc_ref[sl, :], m)

    @pl.when(j == nk - 1)
    def _epilogue():
        qf = q.astype(jnp.float32)
        q_sq = 0.25 * (jnp.sum(qf * qf, axis=1) - 2.0)
        mn = jnp.min(acc_ref[sl, :], axis=1)
        d2 = jnp.maximum(q_sq + mn, 0.0)
        ps = jnp.sqrt(d2 + 1e-12)
        ps_ref[...] = ps.reshape(1, 1, _QB)
        im_ref[...] = jnp.broadcast_to(jnp.max(ps), (1, 1, _LANES))


def kernel(queries, keys):
    Q, D = queries.shape
    K, _ = keys.shape
    n_img = 4
    nq = Q // _QB
    nk = (K + _KB - 1) // _KB
    kp = nk * _KB

    ksq = jnp.sum(keys * keys, axis=1, keepdims=True)
    hi = ksq.astype(jnp.bfloat16).astype(jnp.float32)
    lo = ksq - hi
    feats = jnp.concatenate([keys, hi, lo], axis=1)
    pad = jnp.concatenate(
        [jnp.zeros((kp - K, D), jnp.float32),
         jnp.full((kp - K, 1), 1e30, jnp.float32),
         jnp.zeros((kp - K, 1), jnp.float32)], axis=1)
    kb = jnp.concatenate([feats, pad], axis=0).astype(jnp.bfloat16)
    qb = jnp.concatenate(
        [-2.0 * queries, jnp.ones((Q, 2), jnp.float32)],
        axis=1).astype(jnp.bfloat16)

    ps3, im3 = pl.pallas_call(
        lambda qr, kr, pr, ir, ar: _knn_kernel(qr, kr, pr, ir, ar, nk=nk),
        grid=(nk, nq),
        in_specs=[
            pl.BlockSpec((_QB, _DA), lambda j, i: (i, 0)),
            pl.BlockSpec((_KB, _DA), lambda j, i: (j, 0)),
        ],
        out_specs=[
            pl.BlockSpec((1, 1, _QB), lambda j, i: (i, 0, 0)),
            pl.BlockSpec((1, 1, _LANES), lambda j, i: (i, 0, 0)),
        ],
        out_shape=[
            jax.ShapeDtypeStruct((nq, 1, _QB), jnp.float32),
            jax.ShapeDtypeStruct((n_img, 1, _LANES), jnp.float32),
        ],
        scratch_shapes=[
            pltpu.VMEM((Q, _LANES), jnp.float32),
        ],
    )(qb, kb)

    patch_scores = ps3.reshape(-1)
    image_scores = im3[:, 0, 0].reshape(n_img)
    return image_scores, patch_scores


# speedup vs baseline: 4.0307x; 1.0007x over previous
import jax
import jax.numpy as jnp
from jax.experimental import pallas as pl
from jax.experimental.pallas import tpu as pltpu

_QB = 784
_KB = 1024
_DA = 130
_LANES = 128
_Q = 3136
_NIMG = 4


def _stream_kernel(q_ref, k_ref, acc_ref):
    j = pl.program_id(0)
    i = pl.program_id(1)

    cand = jax.lax.dot_general(
        q_ref[...], k_ref[...], (((1,), (1,)), ((), ())),
        preferred_element_type=jnp.float32)
    m = cand[:, 0:_LANES]
    for c in range(1, _KB // _LANES):
        m = jnp.minimum(m, cand[:, c * _LANES:(c + 1) * _LANES])

    sl = pl.ds(i * _QB, _QB)

    @pl.when(j == 0)
    def _first():
        acc_ref[sl, :] = m

    @pl.when(j > 0)
    def _rest():
        acc_ref[sl, :] = jnp.minimum(acc_ref[sl, :], m)


def _final_kernel(q_ref, acc_ref, ps_ref, im_ref):
    qf = q_ref[...].astype(jnp.float32)
    q_sq = 0.25 * (jnp.sum(qf * qf, axis=1) - 2.0)
    mn = jnp.min(acc_ref[...], axis=1)
    d2 = jnp.maximum(q_sq + mn, 0.0)
    ps = jnp.sqrt(d2 + 1e-12)
    ps2 = ps.reshape(_NIMG, _QB)
    ps_ref[...] = ps2.reshape(_NIMG, 1, _QB)
    imax = jnp.max(ps2, axis=1).reshape(_NIMG, 1, 1)
    im_ref[...] = jnp.broadcast_to(imax, (_NIMG, 1, _LANES))


def kernel(queries, keys):
    Q, D = queries.shape
    K, _ = keys.shape
    nq = Q // _QB
    nk = (K + _KB - 1) // _KB
    kp = nk * _KB

    ksq = jnp.sum(keys * keys, axis=1, keepdims=True)
    hi = ksq.astype(jnp.bfloat16).astype(jnp.float32)
    lo = ksq - hi
    feats = jnp.concatenate([keys, hi, lo], axis=1)
    pad = jnp.concatenate(
        [jnp.zeros((kp - K, D), jnp.float32),
         jnp.full((kp - K, 1), 1e30, jnp.float32),
         jnp.zeros((kp - K, 1), jnp.float32)], axis=1)
    kb = jnp.concatenate([feats, pad], axis=0).astype(jnp.bfloat16)
    qb = jnp.concatenate(
        [-2.0 * queries, jnp.ones((Q, 2), jnp.float32)],
        axis=1).astype(jnp.bfloat16)

    minacc = pl.pallas_call(
        _stream_kernel,
        grid=(nk, nq),
        in_specs=[
            pl.BlockSpec((_QB, _DA), lambda j, i: (i, 0)),
            pl.BlockSpec((_KB, _DA), lambda j, i: (j, 0)),
        ],
        out_specs=pl.BlockSpec((Q, _LANES), lambda j, i: (0, 0)),
        out_shape=jax.ShapeDtypeStruct((Q, _LANES), jnp.float32),
    )(qb, kb)

    ps3, im3 = pl.pallas_call(
        _final_kernel,
        out_shape=[
            jax.ShapeDtypeStruct((_NIMG, 1, _QB), jnp.float32),
            jax.ShapeDtypeStruct((_NIMG, 1, _LANES), jnp.float32),
        ],
    )(qb, minacc)

    patch_scores = ps3.reshape(-1)
    image_scores = im3[:, 0, 0].reshape(_NIMG)
    return image_scores, patch_scores


# speedup vs baseline: 6.3472x; 1.5747x over previous
import jax
import jax.numpy as jnp
from jax.experimental import pallas as pl
from jax.experimental.pallas import tpu as pltpu

_QB = 784
_KB = 1024
_D = 128
_LANES = 128
_NIMG = 4


def _stream_kernel(q_ref, k_ref, acc_ref, kb_ref, ksq_ref, *, kk_total):
    j = pl.program_id(0)
    i = pl.program_id(1)

    @pl.when(i == 0)
    def _load_block():
        rows_left = kk_total - j * _KB
        kf = k_ref[...]
        row_id = jax.lax.broadcasted_iota(jnp.int32, (_KB, _D), 0)
        kf = jnp.where(row_id < rows_left, kf, 0.0)
        kb_ref[...] = kf.astype(jnp.bfloat16)
        kk = (kf * kf).astype(jnp.bfloat16)
        ones = jnp.ones((8, _D), jnp.bfloat16)
        ksq = jax.lax.dot_general(
            ones, kk, (((1,), (1,)), ((), ())),
            preferred_element_type=jnp.float32)
        col_id = jax.lax.broadcasted_iota(jnp.int32, (8, _KB), 1)
        ksq_ref[...] = jnp.where(col_id < rows_left, ksq, 3.0e38)

    cand = jax.lax.dot_general(
        q_ref[...], kb_ref[...], (((1,), (1,)), ((), ())),
        preferred_element_type=jnp.float32)
    cand = cand + ksq_ref[0:1, :]
    m = cand[:, 0:_LANES]
    for c in range(1, _KB // _LANES):
        m = jnp.minimum(m, cand[:, c * _LANES:(c + 1) * _LANES])

    sl = pl.ds(i * _QB, _QB)

    @pl.when(j == 0)
    def _first():
        acc_ref[sl, :] = m

    @pl.when(j > 0)
    def _rest():
        acc_ref[sl, :] = jnp.minimum(acc_ref[sl, :], m)


def _final_kernel(q_ref, acc_ref, ps_ref, im_ref):
    qf = q_ref[...].astype(jnp.float32)
    q_sq = 0.25 * jnp.sum(qf * qf, axis=1)
    mn = jnp.min(acc_ref[...], axis=1)
    d2 = jnp.maximum(q_sq + mn, 0.0)
    ps = jnp.sqrt(d2 + 1e-12)
    ps2 = ps.reshape(_NIMG, _QB)
    ps_ref[...] = ps2.reshape(_NIMG, 1, _QB)
    imax = jnp.max(ps2, axis=1).reshape(_NIMG, 1, 1)
    im_ref[...] = jnp.broadcast_to(imax, (_NIMG, 1, _LANES))


def kernel(queries, keys):
    Q, D = queries.shape
    K, _ = keys.shape
    nq = Q // _QB
    nk = (K + _KB - 1) // _KB

    qb = (-2.0 * queries).astype(jnp.bfloat16)

    minacc = pl.pallas_call(
        lambda qr, kr, ar, kbr, ksr: _stream_kernel(
            qr, kr, ar, kbr, ksr, kk_total=K),
        grid=(nk, nq),
        in_specs=[
            pl.BlockSpec((_QB, D), lambda j, i: (i, 0)),
            pl.BlockSpec((_KB, D), lambda j, i: (j, 0)),
        ],
        out_specs=pl.BlockSpec((Q, _LANES), lambda j, i: (0, 0)),
        out_shape=jax.ShapeDtypeStruct((Q, _LANES), jnp.float32),
        scratch_shapes=[
            pltpu.VMEM((_KB, _D), jnp.bfloat16),
            pltpu.VMEM((8, _KB), jnp.float32),
        ],
    )(qb, keys)

    ps3, im3 = pl.pallas_call(
        _final_kernel,
        out_shape=[
            jax.ShapeDtypeStruct((_NIMG, 1, _QB), jnp.float32),
            jax.ShapeDtypeStruct((_NIMG, 1, _LANES), jnp.float32),
        ],
    )(qb, minacc)

    patch_scores = ps3.reshape(-1)
    image_scores = im3[:, 0, 0].reshape(_NIMG)
    return image_scores, patch_scores


# speedup vs baseline: 11.8178x; 1.8619x over previous
import jax
import jax.numpy as jnp
from jax.experimental import pallas as pl
from jax.experimental.pallas import tpu as pltpu

_QB = 3136
_KB = 2048
_D = 128
_LANES = 128
_NIMG = 4
_PPI = 784


def _stream_kernel(q_ref, k_ref, acc_ref, kb_ref, ksq_ref, *, kk_total):
    j = pl.program_id(0)
    i = pl.program_id(1)

    @pl.when(i == 0)
    def _load_block():
        rows_left = kk_total - j * _KB
        kf = k_ref[...]
        row_id = jax.lax.broadcasted_iota(jnp.int32, (_KB, _D), 0)
        kf = jnp.where(row_id < rows_left, kf, 0.0)
        kb_ref[...] = kf.astype(jnp.bfloat16)
        kk = (kf * kf).astype(jnp.bfloat16)
        ones = jnp.ones((8, _D), jnp.bfloat16)
        ksq = jax.lax.dot_general(
            ones, kk, (((1,), (1,)), ((), ())),
            preferred_element_type=jnp.float32)
        col_id = jax.lax.broadcasted_iota(jnp.int32, (8, _KB), 1)
        ksq_ref[...] = jnp.where(col_id < rows_left, ksq, 3.0e38)

    cand = jax.lax.dot_general(
        q_ref[...], kb_ref[...], (((1,), (1,)), ((), ())),
        preferred_element_type=jnp.float32)
    cand = cand + ksq_ref[0:1, :]
    m = cand[:, 0:_LANES]
    for c in range(1, _KB // _LANES):
        m = jnp.minimum(m, cand[:, c * _LANES:(c + 1) * _LANES])

    sl = pl.ds(i * _QB, _QB)

    @pl.when(j == 0)
    def _first():
        acc_ref[sl, :] = m

    @pl.when(j > 0)
    def _rest():
        acc_ref[sl, :] = jnp.minimum(acc_ref[sl, :], m)


def _final_kernel(q_ref, acc_ref, ps_ref, im_ref):
    qf = q_ref[...].astype(jnp.float32)
    q_sq = 0.25 * jnp.sum(qf * qf, axis=1)
    mn = jnp.min(acc_ref[...], axis=1)
    d2 = jnp.maximum(q_sq + mn, 0.0)
    ps = jnp.sqrt(d2 + 1e-12)
    ps2 = ps.reshape(_NIMG, _PPI)
    ps_ref[...] = ps2.reshape(_NIMG, 1, _PPI)
    imax = jnp.max(ps2, axis=1).reshape(_NIMG, 1, 1)
    im_ref[...] = jnp.broadcast_to(imax, (_NIMG, 1, _LANES))


def kernel(queries, keys):
    Q, D = queries.shape
    K, _ = keys.shape
    nq = Q // _QB
    nk = (K + _KB - 1) // _KB

    qb = (-2.0 * queries).astype(jnp.bfloat16)

    minacc = pl.pallas_call(
        lambda qr, kr, ar, kbr, ksr: _stream_kernel(
            qr, kr, ar, kbr, ksr, kk_total=K),
        grid=(nk, nq),
        in_specs=[
            pl.BlockSpec((_QB, D), lambda j, i: (i, 0)),
            pl.BlockSpec((_KB, D), lambda j, i: (j, 0)),
        ],
        out_specs=pl.BlockSpec((Q, _LANES), lambda j, i: (0, 0)),
        out_shape=jax.ShapeDtypeStruct((Q, _LANES), jnp.float32),
        scratch_shapes=[
            pltpu.VMEM((_KB, _D), jnp.bfloat16),
            pltpu.VMEM((8, _KB), jnp.float32),
        ],
    )(qb, keys)

    ps3, im3 = pl.pallas_call(
        _final_kernel,
        out_shape=[
            jax.ShapeDtypeStruct((_NIMG, 1, _PPI), jnp.float32),
            jax.ShapeDtypeStruct((_NIMG, 1, _LANES), jnp.float32),
        ],
    )(qb, minacc)

    patch_scores = ps3.reshape(-1)
    image_scores = im3[:, 0, 0].reshape(_NIMG)
    return image_scores, patch_scores


# speedup vs baseline: 11.8306x; 1.0011x over previous
import jax
import jax.numpy as jnp
from jax.experimental import pallas as pl
from jax.experimental.pallas import tpu as pltpu

_KB = 2048
_D = 128
_LANES = 128
_NIMG = 4
_PPI = 784


def _stream_kernel(q_ref, k_ref, acc_ref, kb_ref, ksq_ref, *, kk_total):
    j = pl.program_id(0)

    rows_left = kk_total - j * _KB
    kf = k_ref[...]
    row_id = jax.lax.broadcasted_iota(jnp.int32, (_KB, _D), 0)
    kf = jnp.where(row_id < rows_left, kf, 0.0)
    kb_ref[...] = kf.astype(jnp.bfloat16)
    kk = (kf * kf).astype(jnp.bfloat16)
    ones = jnp.ones((8, _D), jnp.bfloat16)
    ksq = jax.lax.dot_general(
        ones, kk, (((1,), (1,)), ((), ())),
        preferred_element_type=jnp.float32)
    col_id = jax.lax.broadcasted_iota(jnp.int32, (8, _KB), 1)
    ksq_ref[...] = jnp.where(col_id < rows_left, ksq, 3.0e38)

    cand = jax.lax.dot_general(
        q_ref[...], kb_ref[...], (((1,), (1,)), ((), ())),
        preferred_element_type=jnp.float32)
    cand = cand + ksq_ref[0:1, :]
    m = cand[:, 0:_LANES]
    for c in range(1, _KB // _LANES):
        m = jnp.minimum(m, cand[:, c * _LANES:(c + 1) * _LANES])

    @pl.when(j == 0)
    def _first():
        acc_ref[...] = m

    @pl.when(j > 0)
    def _rest():
        acc_ref[...] = jnp.minimum(acc_ref[...], m)


def _final_kernel(q_ref, acc_ref, ps_ref, im_ref):
    qf = q_ref[...].astype(jnp.float32)
    q_sq = 0.25 * jnp.sum(qf * qf, axis=1)
    mn = jnp.min(acc_ref[...], axis=1)
    d2 = jnp.maximum(q_sq + mn, 0.0)
    ps = jnp.sqrt(d2 + 1e-12)
    ps2 = ps.reshape(_NIMG, _PPI)
    ps_ref[...] = ps2.reshape(_NIMG, 1, _PPI)
    imax = jnp.max(ps2, axis=1).reshape(_NIMG, 1, 1)
    im_ref[...] = jnp.broadcast_to(imax, (_NIMG, 1, _LANES))


def kernel(queries, keys):
    Q, D = queries.shape
    K, _ = keys.shape
    nk = (K + _KB - 1) // _KB

    qb = (-2.0 * queries).astype(jnp.bfloat16)

    minacc = pl.pallas_call(
        lambda qr, kr, ar, kbr, ksr: _stream_kernel(
            qr, kr, ar, kbr, ksr, kk_total=K),
        grid=(nk,),
        in_specs=[
            pl.BlockSpec((Q, _D), lambda j: (0, 0)),
            pl.BlockSpec((_KB, _D), lambda j: (j, 0)),
        ],
        out_specs=pl.BlockSpec((Q, _LANES), lambda j: (0, 0)),
        out_shape=jax.ShapeDtypeStruct((Q, _LANES), jnp.float32),
        scratch_shapes=[
            pltpu.VMEM((_KB, _D), jnp.bfloat16),
            pltpu.VMEM((8, _KB), jnp.float32),
        ],
    )(qb, keys)

    ps3, im3 = pl.pallas_call(
        _final_kernel,
        out_shape=[
            jax.ShapeDtypeStruct((_NIMG, 1, _PPI), jnp.float32),
            jax.ShapeDtypeStruct((_NIMG, 1, _LANES), jnp.float32),
        ],
    )(qb, minacc)

    patch_scores = ps3.reshape(-1)
    image_scores = im3[:, 0, 0].reshape(_NIMG)
    return image_scores, patch_scores


# speedup vs baseline: 12.1805x; 1.0296x over previous
import jax
import jax.numpy as jnp
from jax.experimental import pallas as pl
from jax.experimental.pallas import tpu as pltpu

_KB = 2048
_NB = 2
_D = 128
_LANES = 128
_NIMG = 4
_PPI = 784


def _prep_block(k_ref, kb_ref, ksq_ref, row0, rows_left):
    kf = k_ref[row0:row0 + _KB, :]
    row_id = jax.lax.broadcasted_iota(jnp.int32, (_KB, _D), 0)
    kf = jnp.where(row_id < rows_left, kf, 0.0)
    kb_ref[...] = kf.astype(jnp.bfloat16)
    kk = (kf * kf).astype(jnp.bfloat16)
    ones = jnp.ones((8, _D), jnp.bfloat16)
    ksq = jax.lax.dot_general(
        ones, kk, (((1,), (1,)), ((), ())),
        preferred_element_type=jnp.float32)
    col_id = jax.lax.broadcasted_iota(jnp.int32, (8, _KB), 1)
    ksq_ref[...] = jnp.where(col_id < rows_left, ksq, 3.0e38)


def _tile_min(q_ref, kb_ref, ksq_ref):
    cand = jax.lax.dot_general(
        q_ref[...], kb_ref[...], (((1,), (1,)), ((), ())),
        preferred_element_type=jnp.float32)
    cand = cand + ksq_ref[0:1, :]
    m = cand[:, 0:_LANES]
    for c in range(1, _KB // _LANES):
        m = jnp.minimum(m, cand[:, c * _LANES:(c + 1) * _LANES])
    return m


def _stream_kernel(q_ref, k_ref, acc_ref, kba, ksqa, kbb, ksqb, *, kk_total):
    j = pl.program_id(0)
    base = j * (_NB * _KB)

    _prep_block(k_ref, kba, ksqa, 0, kk_total - base)
    _prep_block(k_ref, kbb, ksqb, _KB, kk_total - base - _KB)
    ma = _tile_min(q_ref, kba, ksqa)
    mb = _tile_min(q_ref, kbb, ksqb)
    m = jnp.minimum(ma, mb)

    @pl.when(j == 0)
    def _first():
        acc_ref[...] = m

    @pl.when(j > 0)
    def _rest():
        acc_ref[...] = jnp.minimum(acc_ref[...], m)


def _final_kernel(q_ref, acc_ref, ps_ref, im_ref):
    qf = q_ref[...].astype(jnp.float32)
    q_sq = 0.25 * jnp.sum(qf * qf, axis=1)
    mn = jnp.min(acc_ref[...], axis=1)
    d2 = jnp.maximum(q_sq + mn, 0.0)
    ps = jnp.sqrt(d2 + 1e-12)
    ps2 = ps.reshape(_NIMG, _PPI)
    ps_ref[...] = ps2.reshape(_NIMG, 1, _PPI)
    imax = jnp.max(ps2, axis=1).reshape(_NIMG, 1, 1)
    im_ref[...] = jnp.broadcast_to(imax, (_NIMG, 1, _LANES))


def kernel(queries, keys):
    Q, D = queries.shape
    K, _ = keys.shape
    step = _NB * _KB
    nj = (K + step - 1) // step

    qb = (-2.0 * queries).astype(jnp.bfloat16)

    minacc = pl.pallas_call(
        lambda qr, kr, ar, a1, a2, b1, b2: _stream_kernel(
            qr, kr, ar, a1, a2, b1, b2, kk_total=K),
        grid=(nj,),
        in_specs=[
            pl.BlockSpec((Q, _D), lambda j: (0, 0)),
            pl.BlockSpec((step, _D), lambda j: (j, 0)),
        ],
        out_specs=pl.BlockSpec((Q, _LANES), lambda j: (0, 0)),
        out_shape=jax.ShapeDtypeStruct((Q, _LANES), jnp.float32),
        scratch_shapes=[
            pltpu.VMEM((_KB, _D), jnp.bfloat16),
            pltpu.VMEM((8, _KB), jnp.float32),
            pltpu.VMEM((_KB, _D), jnp.bfloat16),
            pltpu.VMEM((8, _KB), jnp.float32),
        ],
    )(qb, keys)

    ps3, im3 = pl.pallas_call(
        _final_kernel,
        out_shape=[
            jax.ShapeDtypeStruct((_NIMG, 1, _PPI), jnp.float32),
            jax.ShapeDtypeStruct((_NIMG, 1, _LANES), jnp.float32),
        ],
    )(qb, minacc)

    patch_scores = ps3.reshape(-1)
    image_scores = im3[:, 0, 0].reshape(_NIMG)
    return image_scores, patch_scores
